# Initial kernel scaffold; baseline (speedup 1.0000x reference)
import jax
import jax.numpy as jnp
from jax.experimental import pallas as pl


def kernel(rays, densities, rgb):
    raise NotImplementedError("write your pallas kernel here")


# speedup vs baseline: 2.2592x; 2.2592x over previous
import functools

import jax
import jax.numpy as jnp
from jax import lax
from jax.experimental import pallas as pl
from jax.experimental.pallas import tpu as pltpu
from jax.experimental.pallas import tpu_sc as plsc

RESO = 128
OUT = 3
G_RAD = 1.3
T_NEAR = 0.2
T_FAR = 2.0
STEPS = 64
VOXEL_LEN = G_RAD * 2 / RESO
N_RAYS = 4096
EPS = 1e-10

P = STEPS * N_RAYS
NW = 32
PPW = P // NW
CH = 512
NCHUNK = PPW // CH
NBLK = CH // 128


def _tc_prep(rays_t, ts2):
    JB = 8

    def body(rays_ref, ts_ref, base_ref, slot_ref, w_ref):
        t = ts_ref[...]
        pts = []
        for d in range(3):
            ro = rays_ref[d:d + 1, :]
            rd = rays_ref[d + 3:d + 4, :]
            pts.append(ro + t * rd)
        ilo, ihi, tx = [], [], []
        for d in range(3):
            p = pts[d]
            nlo = jnp.clip(-0.5 * VOXEL_LEN + p, -G_RAD, G_RAD)
            nhi = jnp.clip(0.5 * VOXEL_LEN + p, -G_RAD, G_RAD)
            clo = jnp.clip((jnp.floor(nlo / VOXEL_LEN + EPS) + 0.5) * VOXEL_LEN,
                           -(G_RAD - VOXEL_LEN / 2), G_RAD - VOXEL_LEN / 2)
            chi = jnp.clip((jnp.floor(nhi / VOXEL_LEN + EPS) + 0.5) * VOXEL_LEN,
                           -(G_RAD - VOXEL_LEN / 2), G_RAD - VOXEL_LEN / 2)
            ilo.append(jnp.floor(clo / VOXEL_LEN + EPS).astype(jnp.int32) + RESO // 2)
            ihi.append(jnp.floor(chi / VOXEL_LEN + EPS).astype(jnp.int32) + RESO // 2)
            x = (p - clo) / VOXEL_LEN
            tx.append((1 - x, x))
        zbase = jnp.minimum(ilo[2], RESO - 2)
        for cu in range(4):
            bx, by = cu & 1, (cu >> 1) & 1
            ix = ihi[0] if bx else ilo[0]
            iy = ihi[1] if by else ilo[1]
            base_ref[cu] = (ix * RESO + iy) * RESO + zbase
        for u in range(8):
            bx, by, bz = u & 1, (u >> 1) & 1, (u >> 2) & 1
            iz = ihi[2] if bz else ilo[2]
            slot_ref[u] = jnp.where(iz == zbase, 0, 4).astype(jnp.int32)
            w_ref[u] = tx[0][bx] * tx[1][by] * tx[2][bz]

    return pl.pallas_call(
        body,
        grid=(STEPS // JB,),
        in_specs=[
            pl.BlockSpec((6, N_RAYS), lambda j: (0, 0)),
            pl.BlockSpec((JB, 1), lambda j: (j, 0)),
        ],
        out_specs=[
            pl.BlockSpec((4, JB, N_RAYS), lambda j: (0, j, 0)),
            pl.BlockSpec((8, JB, N_RAYS), lambda j: (0, j, 0)),
            pl.BlockSpec((8, JB, N_RAYS), lambda j: (0, j, 0)),
        ],
        out_shape=[
            jax.ShapeDtypeStruct((4, STEPS, N_RAYS), jnp.int32),
            jax.ShapeDtypeStruct((8, STEPS, N_RAYS), jnp.int32),
            jax.ShapeDtypeStruct((8, STEPS, N_RAYS), jnp.float32),
        ],
    )(rays_t, ts2)


def _sc_gather(table2, base4, slot8, w8):
    mesh = plsc.VectorSubcoreMesh(core_axis_name="c", subcore_axis_name="s")

    @functools.partial(
        pl.kernel,
        mesh=mesh,
        compiler_params=pltpu.CompilerParams(
            needs_layout_passes=False, use_tc_tiling_on_sc=False),
        out_type=jax.ShapeDtypeStruct((4, P), jnp.float32),
        scratch_types=[
            pltpu.VMEM((4, NBLK, 128), jnp.int32),
            pltpu.VMEM((8, NBLK, 128), jnp.int32),
            pltpu.VMEM((8, NBLK, 128), jnp.float32),
            pltpu.VMEM((4, NBLK, 128, 8), jnp.float32),
            pltpu.VMEM((4, CH), jnp.float32),
            pltpu.SemaphoreType.DMA,
        ],
    )
    def k(tab_hbm, base_hbm, slot_hbm, w_hbm, out_hbm,
          base_v, slot_v, w_v, rows_v, out_v, gsem):
        wid = lax.axis_index("s") * 2 + lax.axis_index("c")
        base_blk = wid * (PPW // 128)
        iota = lax.iota(jnp.int32, 16)
        lvecs = [iota + m * 16 for m in range(8)]

        def chunk(ci, carry):
            blk = base_blk + ci * NBLK
            pltpu.sync_copy(base_hbm.at[:, pl.ds(blk, NBLK)], base_v)
            pltpu.sync_copy(slot_hbm.at[:, pl.ds(blk, NBLK)], slot_v)
            pltpu.sync_copy(w_hbm.at[:, pl.ds(blk, NBLK)], w_v)
            handles = []
            for cu in range(4):
                for kb in range(NBLK):
                    handles.append(pltpu.async_copy(
                        tab_hbm.at[base_v.at[cu, kb]], rows_v.at[cu, kb], gsem))
            for h in handles:
                h.wait()
            for g in range(CH // 16):
                kb = g // 8
                lvec = lvecs[g % 8]
                kbv = jnp.full((16,), kb, jnp.int32)
                accs = [jnp.zeros((16,), jnp.float32) for _ in range(4)]
                for u in range(8):
                    uv = jnp.full((16,), u, jnp.int32)
                    cuv = jnp.full((16,), u & 3, jnp.int32)
                    slotv = plsc.load_gather(slot_v, [uv, kbv, lvec])
                    wv = plsc.load_gather(w_v, [uv, kbv, lvec])
                    for c in range(4):
                        val = plsc.load_gather(
                            rows_v, [cuv, kbv, lvec, slotv + c])
                        prod = wv * val
                        accs[c] = accs[c] + prod
                for c in range(4):
                    out_v[c, pl.ds(g * 16, 16)] = accs[c]
            pt0 = wid * PPW + ci * CH
            pltpu.sync_copy(out_v, out_hbm.at[:, pl.ds(pt0, CH)])
            return carry

        lax.fori_loop(0, NCHUNK, chunk, 0)

    return k(table2, base4, slot8, w8)


def _tc_finish(acc4, rays_t, dt):
    JB = 8

    def body(acc_ref, rays_ref, dt_ref, o_ref):
        dens = acc_ref[0]
        rx = rays_ref[3:4, :]
        ry = rays_ref[4:5, :]
        rz = rays_ref[5:6, :]
        norm = jnp.sqrt(rx * rx + ry * ry + rz * rz)
        dt_s = dt_ref[0, 0]
        sigma_a = jax.nn.softplus(dens - 1)
        a = 1 - jnp.exp(-sigma_a * (dt_s * norm))
        b = 1 - jnp.exp(-sigma_a * (1e10 * norm))
        q = 1 - a + 1e-10
        q2 = q * q
        q4 = q2 * q2
        q8 = q4 * q4
        q16 = q8 * q8
        q32 = q16 * q16
        q63 = q32 * q16 * q8 * q4 * q2 * q
        s63 = jnp.where(jnp.abs(1 - q) > 1e-9, (1 - q63) / (1 - q), 63.0)
        wtot = a * s63 + b * q63
        for c in range(3):
            o_ref[c] = wtot * acc_ref[1 + c]

    return pl.pallas_call(
        body,
        grid=(STEPS // JB,),
        in_specs=[
            pl.BlockSpec((4, JB, N_RAYS), lambda j: (0, j, 0)),
            pl.BlockSpec((6, N_RAYS), lambda j: (0, 0)),
            pl.BlockSpec((1, 1), lambda j: (0, 0), memory_space=pltpu.SMEM),
        ],
        out_specs=pl.BlockSpec((3, JB, N_RAYS), lambda j: (0, j, 0)),
        out_shape=jax.ShapeDtypeStruct((3, STEPS, N_RAYS), jnp.float32),
    )(acc4, rays_t, dt.reshape(1, 1))


def kernel(rays, densities, rgb):
    ts = jnp.linspace(T_NEAR, T_FAR, STEPS, dtype=rays.dtype)
    dt = jnp.clip(ts[1] - ts[0], 1e-5, None)
    rays_t = rays.T
    act_rgb = jax.nn.sigmoid(rgb) * (1 + 2e-3) - 1e-3
    d2 = densities.reshape(-1, 1)
    a2 = act_rgb.reshape(-1, OUT)
    table2 = jnp.concatenate([d2[:-1], a2[:-1], d2[1:], a2[1:]], axis=-1)

    base4, slot8, w8 = _tc_prep(rays_t, ts[:, None])
    acc = _sc_gather(table2,
                     base4.reshape(4, P // 128, 128),
                     slot8.reshape(8, P // 128, 128),
                     w8.reshape(8, P // 128, 128))
    o3 = _tc_finish(acc.reshape(4, STEPS, N_RAYS), rays_t, dt)
    return jnp.transpose(o3, (1, 2, 0))[None]
